# baseline (speedup 1.0000x reference)
import jax
import jax.numpy as jnp
from jax import lax
from jax.experimental import pallas as pl
from jax.experimental.pallas import tpu as pltpu
from jax.experimental.pallas import tpu_sc as plsc

_NSRC = 32
_NY = 1024
_NX = 2048
_L = 16


def _sc_body(yflat, y_hbm, x_hbm, upd_hbm, yv, xv, updv, idxv, elemv, sem):
    cid = lax.axis_index("c")
    sid = lax.axis_index("s")

    @pl.when(jnp.logical_and(cid == 0, sid == 0))
    def _():
        pltpu.sync_copy(y_hbm, yv)
        pltpu.sync_copy(x_hbm, xv)
        pltpu.sync_copy(upd_hbm, updv)
        for c in range(_NSRC // _L):
            shot = lax.iota(jnp.int32, _L) + jnp.int32(c * _L)
            yc = yv[pl.ds(c * _L, _L)]
            xc = xv[pl.ds(c * _L, _L)]
            idxv[pl.ds(c * _L, _L)] = (
                shot * jnp.int32(_NY * _NX) + yc * jnp.int32(_NX) + xc
            )
        pltpu.async_copy(yflat.at[idxv], elemv, sem).wait()
        for c in range(_NSRC // _L):
            sl = pl.ds(c * _L, _L)
            elemv[sl] = elemv[sl] + updv[...]
        pltpu.async_copy(elemv, yflat.at[idxv], sem).wait()


_scatter_add = pl.kernel(
    _sc_body,
    out_type=(),
    mesh=plsc.VectorSubcoreMesh(core_axis_name="c", subcore_axis_name="s"),
    scratch_types=[
        pltpu.VMEM((_NSRC,), jnp.int32),
        pltpu.VMEM((_NSRC,), jnp.int32),
        pltpu.VMEM((_L,), jnp.float32),
        pltpu.VMEM((_NSRC,), jnp.int32),
        pltpu.VMEM((_NSRC,), jnp.float32),
        pltpu.SemaphoreType.DMA,
    ],
)


def kernel(Y, X, y, x, dt=1.0):
    upd = jnp.asarray(dt, jnp.float32) * X.astype(jnp.float32).reshape(())
    upd16 = jnp.broadcast_to(upd, (_L,))
    yref = jax.new_ref(Y.reshape(_NSRC * _NY * _NX))
    _scatter_add(yref, y.astype(jnp.int32), x.astype(jnp.int32), upd16)
    return yref[...].reshape(_NSRC, _NY, _NX)


# speedup vs baseline: 1.0001x; 1.0001x over previous
import jax
import jax.numpy as jnp
from jax import lax
from jax.experimental import pallas as pl
from jax.experimental.pallas import tpu as pltpu
from jax.experimental.pallas import tpu_sc as plsc

_NSRC = 32
_NY = 1024
_NX = 2048
_L = 16


def _sc_body(yflat, y_hbm, x_hbm, upd_hbm, yv, xv, updv, idxv, elemv, sem):
    cid = lax.axis_index("c")
    sid = lax.axis_index("s")

    @pl.when(jnp.logical_and(cid == 0, sid == 0))
    def _():
        pltpu.sync_copy(y_hbm, yv)
        pltpu.sync_copy(x_hbm, xv)
        pltpu.sync_copy(upd_hbm, updv)
        for c in range(_NSRC // _L):
            shot = lax.iota(jnp.int32, _L) + jnp.int32(c * _L)
            yc = yv[pl.ds(c * _L, _L)]
            xc = xv[pl.ds(c * _L, _L)]
            idxv[pl.ds(c * _L, _L)] = (
                shot * jnp.int32(_NY * _NX) + yc * jnp.int32(_NX) + xc
            )
        pltpu.async_copy(yflat.at[idxv], elemv, sem).wait()
        for c in range(_NSRC // _L):
            sl = pl.ds(c * _L, _L)
            elemv[sl] = elemv[sl] + updv[...]
        pltpu.async_copy(elemv, yflat.at[idxv], sem).wait()


_scatter_add = pl.kernel(
    _sc_body,
    out_type=(),
    mesh=plsc.VectorSubcoreMesh(core_axis_name="c", subcore_axis_name="s"),
    scratch_types=[
        pltpu.VMEM((_NSRC,), jnp.int32),
        pltpu.VMEM((_NSRC,), jnp.int32),
        pltpu.VMEM((_L,), jnp.float32),
        pltpu.VMEM((_NSRC,), jnp.int32),
        pltpu.VMEM((_NSRC,), jnp.float32),
        pltpu.SemaphoreType.DMA,
    ],
)


def kernel(Y, X, y, x, dt=1.0):
    upd = jnp.asarray(dt, jnp.float32) * X.astype(jnp.float32).reshape(())
    upd16 = jnp.broadcast_to(upd, (_L,))
    yref = jax.new_ref(Y.reshape(_NSRC * _NY * _NX))
    _scatter_add(yref, y.astype(jnp.int32), x.astype(jnp.int32), upd16)
    return jax.freeze(yref).reshape(_NSRC, _NY, _NX)


# speedup vs baseline: 2.5293x; 2.5291x over previous
import jax
import jax.numpy as jnp
from jax import lax
from jax.experimental import pallas as pl
from jax.experimental.pallas import tpu as pltpu
from jax.experimental.pallas import tpu_sc as plsc

_NSRC = 32
_NY = 1024
_NX = 2048
_L = 16


def _sc_body(yref, y_hbm, x_hbm, upd_hbm, yv, xv, updv, tile, sem):
    cid = lax.axis_index("c")
    sid = lax.axis_index("s")
    wid = sid * 2 + cid

    pltpu.sync_copy(y_hbm, yv)
    pltpu.sync_copy(x_hbm, xv)
    pltpu.sync_copy(upd_hbm, updv)

    lanes = lax.iota(jnp.int32, _L)
    zero = jnp.zeros((_L,), jnp.int32)
    yi = jnp.int32(0)
    xi = jnp.int32(0)
    for c in range(_NSRC // _L):
        m = (lanes + c * _L) == wid
        yi = yi + jnp.sum(jnp.where(m, yv[pl.ds(c * _L, _L)], zero))
        xi = xi + jnp.sum(jnp.where(m, xv[pl.ds(c * _L, _L)], zero))
    row0 = wid * _NY + (yi >> 3) * 8
    col0 = (xi >> 7) * 128
    ry = yi & 7
    c0 = (xi & 127) & ~15
    lane = xi & 15

    pltpu.async_copy(yref.at[pl.ds(row0, 8), pl.ds(col0, 128)], tile, sem).wait()
    sel = lax.iota(jnp.int32, _L) == lane
    delta = jnp.where(sel, updv[...], jnp.float32(0.0))
    tile[ry, pl.ds(c0, _L)] = tile[ry, pl.ds(c0, _L)] + delta
    pltpu.async_copy(tile, yref.at[pl.ds(row0, 8), pl.ds(col0, 128)], sem).wait()


_scatter_add = pl.kernel(
    _sc_body,
    out_type=(),
    mesh=plsc.VectorSubcoreMesh(core_axis_name="c", subcore_axis_name="s"),
    scratch_types=[
        pltpu.VMEM((_NSRC,), jnp.int32),
        pltpu.VMEM((_NSRC,), jnp.int32),
        pltpu.VMEM((_L,), jnp.float32),
        pltpu.VMEM((8, 128), jnp.float32),
        pltpu.SemaphoreType.DMA,
    ],
    compiler_params=pltpu.CompilerParams(
        use_tc_tiling_on_sc=True, needs_layout_passes=False
    ),
)


def kernel(Y, X, y, x, dt=1.0):
    upd = jnp.asarray(dt, jnp.float32) * X.astype(jnp.float32).reshape(())
    upd16 = jnp.broadcast_to(upd, (_L,))
    yref = jax.new_ref(Y.reshape(_NSRC * _NY, _NX))
    _scatter_add(yref, y.astype(jnp.int32), x.astype(jnp.int32), upd16)
    return jax.freeze(yref).reshape(_NSRC, _NY, _NX)
